# initial kernel scaffold (unmeasured)
import jax
import jax.numpy as jnp
from jax import lax
from jax.experimental import pallas as pl
from jax.experimental.pallas import tpu as pltpu

P = 16


def kernel(x, w_mat, scale_x, scale_w):
    M, K = x.shape
    N = w_mat.shape[1]
    NB = N // P

    def body(x_ref, w_ref, sx_ref, sw_ref, out_ref,
             xb_ref, wbuf, sendbuf, copy_sems, send_sems, recv_sems):
        my = lax.axis_index("i")
        s = sx_ref[0] * sw_ref[0]

        xb_ref[...] = x_ref[...].astype(jnp.bfloat16)

        def w_copy(d, slot):
            t = lax.rem(my + d, P)
            return pltpu.make_async_copy(
                w_ref.at[:, pl.ds(t * NB, NB)], wbuf.at[slot],
                copy_sems.at[slot],
            )

        def a2a_rdma(d):
            t = lax.rem(my + d, P)
            return pltpu.make_async_remote_copy(
                src_ref=sendbuf.at[d - 1],
                dst_ref=out_ref.at[pl.ds(my * M, M), :],
                send_sem=send_sems.at[d - 1],
                recv_sem=recv_sems.at[d - 1],
                device_id=(t,),
                device_id_type=pl.DeviceIdType.MESH,
            )

        w_copy(0, 0).start()
        for d in range(P):
            if d + 1 < P:
                w_copy(d + 1, (d + 1) % 2).start()
            w_copy(d, d % 2).wait()
            wb = wbuf[d % 2].astype(jnp.bfloat16)
            blk = jnp.dot(xb_ref[...], wb, preferred_element_type=jnp.float32)
            blk = jnp.maximum(blk * s, 0.0)
            if d == 0:
                out_ref[pl.ds(my * M, M), :] = blk
            else:
                sendbuf[d - 1] = blk
                a2a_rdma(d).start()

        for d in range(1, P):
            a2a_rdma(d).wait_send()
        for d in range(1, P):
            a2a_rdma(d).wait_recv()

    return pl.pallas_call(
        body,
        out_shape=jax.ShapeDtypeStruct((P * M, NB), jnp.float32),
        in_specs=[
            pl.BlockSpec(memory_space=pltpu.VMEM),
            pl.BlockSpec(memory_space=pltpu.ANY),
            pl.BlockSpec(memory_space=pltpu.SMEM),
            pl.BlockSpec(memory_space=pltpu.SMEM),
        ],
        out_specs=pl.BlockSpec(memory_space=pltpu.VMEM),
        scratch_shapes=[
            pltpu.VMEM((M, K), jnp.bfloat16),
            pltpu.VMEM((2, K, NB), w_mat.dtype),
            pltpu.VMEM((P - 1, M, NB), jnp.float32),
            pltpu.SemaphoreType.DMA((2,)),
            pltpu.SemaphoreType.DMA((P - 1,)),
            pltpu.SemaphoreType.DMA((P - 1,)),
        ],
    )(x, w_mat, scale_x, scale_w)


# baseline (device time: 110556 ns/iter reference)
import jax
import jax.numpy as jnp
from jax import lax
from jax.experimental import pallas as pl
from jax.experimental.pallas import tpu as pltpu

P = 16


def kernel(x, w_mat, scale_x, scale_w):
    M, K = x.shape
    N = w_mat.shape[1]
    NB = N // P

    def body(x_ref, w_ref, sx_ref, sw_ref, out_ref,
             xb_ref, wbuf, sendbuf, copy_sems, send_sems, recv_sems):
        my = lax.axis_index("i")
        s = sx_ref[0] * sw_ref[0]

        xb_ref[...] = x_ref[...].astype(jnp.bfloat16)

        def w_copy(d, slot):
            t = lax.rem(my + d, P)
            return pltpu.make_async_copy(
                w_ref.at[:, pl.ds(t * NB, NB)], wbuf.at[slot],
                copy_sems.at[slot],
            )

        def a2a_rdma(d):
            t = lax.rem(my + d, P)
            return pltpu.make_async_remote_copy(
                src_ref=sendbuf.at[d - 1],
                dst_ref=out_ref.at[pl.ds(my * M, M), :],
                send_sem=send_sems.at[d - 1],
                recv_sem=recv_sems.at[d - 1],
                device_id=(t,),
                device_id_type=pl.DeviceIdType.MESH,
            )

        w_copy(0, 0).start()
        for d in range(P):
            if d + 1 < P:
                w_copy(d + 1, (d + 1) % 2).start()
            w_copy(d, d % 2).wait()
            wb = wbuf[d % 2].astype(jnp.bfloat16)
            blk = jnp.dot(xb_ref[...], wb, preferred_element_type=jnp.float32)
            blk = jnp.maximum(blk * s, 0.0)
            if d == 0:
                out_ref[pl.ds(my * M, M), :] = blk
            else:
                sendbuf[d - 1] = blk
                a2a_rdma(d).start()

        for d in range(1, P):
            a2a_rdma(d).wait_send()
        for d in range(1, P):
            a2a_rdma(d).wait_recv()

    return pl.pallas_call(
        body,
        out_shape=jax.ShapeDtypeStruct((P * M, NB), jnp.float32),
        in_specs=[
            pl.BlockSpec(memory_space=pltpu.VMEM),
            pl.BlockSpec(memory_space=pl.ANY),
            pl.BlockSpec(memory_space=pltpu.SMEM),
            pl.BlockSpec(memory_space=pltpu.SMEM),
        ],
        out_specs=pl.BlockSpec(memory_space=pltpu.VMEM),
        scratch_shapes=[
            pltpu.VMEM((M, K), jnp.bfloat16),
            pltpu.VMEM((2, K, NB), w_mat.dtype),
            pltpu.VMEM((P - 1, M, NB), jnp.float32),
            pltpu.SemaphoreType.DMA((2,)),
            pltpu.SemaphoreType.DMA((P - 1,)),
            pltpu.SemaphoreType.DMA((P - 1,)),
        ],
    )(x, w_mat, scale_x, scale_w)


# device time: 74306 ns/iter; 1.4878x vs baseline; 1.4878x over previous
import jax
import jax.numpy as jnp
from jax import lax
from jax.experimental import pallas as pl
from jax.experimental.pallas import tpu as pltpu

P = 16


def kernel(x, w_mat, scale_x, scale_w):
    M, K = x.shape
    N = w_mat.shape[1]
    NB = N // P

    def body(x_ref, w_ref, sx_ref, sw_ref, out_ref,
             xb_ref, wbuf, sendbuf, recvbuf, copy_sems, send_sems,
             recv_sems):
        my = lax.axis_index("i")
        s = sx_ref[0] * sw_ref[0]

        xb_ref[...] = x_ref[...].astype(jnp.bfloat16)

        def w_copy(d, slot):
            t = lax.rem(my + d, P)
            return pltpu.make_async_copy(
                w_ref.at[:, pl.ds(t * NB, NB)], wbuf.at[slot],
                copy_sems.at[slot],
            )

        def a2a_rdma(d):
            t = lax.rem(my + d, P)
            return pltpu.make_async_remote_copy(
                src_ref=sendbuf.at[d - 1],
                dst_ref=recvbuf.at[d - 1],
                send_sem=send_sems.at[d - 1],
                recv_sem=recv_sems.at[d - 1],
                device_id=(t,),
                device_id_type=pl.DeviceIdType.MESH,
            )

        w_copy(0, 0).start()
        for d in range(P):
            if d + 1 < P:
                w_copy(d + 1, (d + 1) % 2).start()
            w_copy(d, d % 2).wait()
            wb = wbuf[d % 2].astype(jnp.bfloat16)
            blk = jnp.dot(xb_ref[...], wb, preferred_element_type=jnp.float32)
            blk = jnp.maximum(blk * s, 0.0)
            if d == 0:
                out_ref[pl.ds(my * M, M), :] = blk
            else:
                sendbuf[d - 1] = blk.astype(jnp.bfloat16)
                a2a_rdma(d).start()

        for d in range(1, P):
            src = lax.rem(my - d + P, P)
            a2a_rdma(d).wait_recv()
            out_ref[pl.ds(src * M, M), :] = recvbuf[d - 1].astype(jnp.float32)
        for d in range(1, P):
            a2a_rdma(d).wait_send()

    return pl.pallas_call(
        body,
        out_shape=jax.ShapeDtypeStruct((P * M, NB), jnp.float32),
        in_specs=[
            pl.BlockSpec(memory_space=pltpu.VMEM),
            pl.BlockSpec(memory_space=pl.ANY),
            pl.BlockSpec(memory_space=pltpu.SMEM),
            pl.BlockSpec(memory_space=pltpu.SMEM),
        ],
        out_specs=pl.BlockSpec(memory_space=pltpu.VMEM),
        scratch_shapes=[
            pltpu.VMEM((M, K), jnp.bfloat16),
            pltpu.VMEM((2, K, NB), w_mat.dtype),
            pltpu.VMEM((P - 1, M, NB), jnp.bfloat16),
            pltpu.VMEM((P - 1, M, NB), jnp.bfloat16),
            pltpu.SemaphoreType.DMA((2,)),
            pltpu.SemaphoreType.DMA((P - 1,)),
            pltpu.SemaphoreType.DMA((P - 1,)),
        ],
    )(x, w_mat, scale_x, scale_w)
